# Initial kernel scaffold; baseline (speedup 1.0000x reference)
import jax
import jax.numpy as jnp
from jax.experimental import pallas as pl


def kernel(latent, W1, b1, Wp, bp, Wv, bv, mem_keys, mem_values):
    raise NotImplementedError("write your pallas kernel here")


# speedup vs baseline: 3.0735x; 3.0735x over previous
import jax
import jax.numpy as jnp
from jax import lax
from jax.experimental import pallas as pl
from jax.experimental.pallas import tpu as pltpu

B, D, A, M, K = 512, 256, 18, 50000, 50
DELTA = 0.001
MP = 51200
CB = 2048
RB = 64
NR, NC = B // RB, MP // CB
INF_BITS = 0x7F800000


def _mlp_body(lat_ref, w1_ref, b1_ref, wp_ref, bp_ref, wv_ref, bv_ref,
              pol_ref, emb_ref):
    out = jnp.maximum(
        jnp.dot(lat_ref[...], w1_ref[...],
                preferred_element_type=jnp.float32) + b1_ref[...], 0.0)
    pol_ref[...] = jnp.dot(out, wp_ref[...],
                           preferred_element_type=jnp.float32) + bp_ref[...]
    emb_ref[...] = jnp.dot(out, wv_ref[...],
                           preferred_element_type=jnp.float32) + bv_ref[...]


def _knn_body(emb_ref, keys_ref, mv_ref, num_ref, ksum_ref, d2_ref):
    c = pl.program_id(1)
    qb = emb_ref[...]
    kb = keys_ref[...]
    cross = lax.dot_general(qb, kb, (((1,), (1,)), ((), ())),
                            preferred_element_type=jnp.float32)
    q2 = jnp.sum(qb * qb, axis=1, keepdims=True)
    k2 = jnp.sum(kb * kb, axis=1)[None, :]
    d2_ref[:, pl.ds(c * CB, CB)] = jnp.maximum(q2 + k2 - 2.0 * cross, 0.0)

    @pl.when(c == NC - 1)
    def _select_and_combine():
        d2 = d2_ref[...]
        cols = lax.broadcasted_iota(jnp.int32, (RB, MP), 1)
        bits = lax.bitcast_convert_type(d2, jnp.int32)
        bits = jnp.where(cols < M, bits, jnp.int32(INF_BITS))

        def bs_body(_, lohi):
            lo, hi = lohi
            mid = lo + lax.div(hi - lo, 2)
            cnt = jnp.sum(jnp.where(bits <= mid, 1, 0), axis=1, keepdims=True)
            ge = cnt >= K
            return (jnp.where(ge, lo, mid + 1), jnp.where(ge, mid, hi))

        lo0 = jnp.zeros((RB, 1), jnp.int32)
        hi0 = jnp.full((RB, 1), INF_BITS, jnp.int32)
        _, t_bits = lax.fori_loop(0, 31, bs_body, (lo0, hi0))

        eq = bits == t_bits
        cnt_lt = jnp.sum(jnp.where(bits < t_bits, 1, 0), axis=1,
                         keepdims=True)
        n_eq = K - cnt_lt

        def bs_idx(_, lohi):
            lo, hi = lohi
            mid = lo + lax.div(hi - lo, 2)
            cnt = jnp.sum(jnp.where(eq & (cols <= mid), 1, 0), axis=1,
                          keepdims=True)
            ge = cnt >= n_eq
            return (jnp.where(ge, lo, mid + 1), jnp.where(ge, mid, hi))

        _, cut = lax.fori_loop(0, 17, bs_idx,
                               (jnp.zeros((RB, 1), jnp.int32),
                                jnp.full((RB, 1), MP, jnp.int32)))

        sel = (bits < t_bits) | (eq & (cols <= cut))
        dist = jnp.sqrt(jnp.maximum(d2, 1e-12))
        kern = jnp.where(sel, 1.0 / (dist + DELTA), 0.0)
        mv = mv_ref[...]
        num_ref[...] = jnp.sum(kern * mv, axis=1, keepdims=True)
        ksum_ref[...] = jnp.sum(kern, axis=1, keepdims=True)


def _norm_body(num_ref, ksum_ref, val_ref):
    val_ref[...] = num_ref[...] / jnp.sum(ksum_ref[...])


def kernel(latent, W1, b1, Wp, bp, Wv, bv, mem_keys, mem_values):
    policy, emb = pl.pallas_call(
        _mlp_body,
        out_shape=[jax.ShapeDtypeStruct((B, A), jnp.float32),
                   jax.ShapeDtypeStruct((B, D), jnp.float32)],
    )(latent, W1, b1.reshape(1, D), Wp, bp.reshape(1, A), Wv,
      bv.reshape(1, D))

    keys_pad = jnp.pad(mem_keys, ((0, MP - M), (0, 0)))
    mv_pad = jnp.pad(mem_values, (0, MP - M)).reshape(1, MP)

    num, ksum = pl.pallas_call(
        _knn_body,
        grid=(NR, NC),
        in_specs=[
            pl.BlockSpec((RB, D), lambda r, c: (r, 0)),
            pl.BlockSpec((CB, D), lambda r, c: (c, 0)),
            pl.BlockSpec((1, MP), lambda r, c: (0, 0)),
        ],
        out_specs=[
            pl.BlockSpec((RB, 1), lambda r, c: (r, 0)),
            pl.BlockSpec((RB, 1), lambda r, c: (r, 0)),
        ],
        out_shape=[jax.ShapeDtypeStruct((B, 1), jnp.float32),
                   jax.ShapeDtypeStruct((B, 1), jnp.float32)],
        scratch_shapes=[pltpu.VMEM((RB, MP), jnp.float32)],
    )(emb, keys_pad, mv_pad)

    value = pl.pallas_call(
        _norm_body,
        out_shape=jax.ShapeDtypeStruct((B, 1), jnp.float32),
    )(num, ksum)

    return policy, value.reshape(B)


# speedup vs baseline: 4.7344x; 1.5404x over previous
import functools

import jax
import jax.numpy as jnp
from jax import lax
from jax.experimental import pallas as pl
from jax.experimental.pallas import tpu as pltpu
from jax.experimental.pallas import tpu_sc as plsc

B, D, A, M, K = 512, 256, 18, 50000, 50
DELTA = 0.001
MP = 51200
CB = 2048
RB = 64
CHW = 128
NCH = MP // CHW
NRB, NCB = B // RB, MP // CB
NC_SC, NS_SC = 2, 16
NW = NC_SC * NS_SC
RPW = B // NW
CAP = 96
INF_BITS = 0x7F800000


def _mlp_body(lat_ref, w1_ref, b1_ref, wp_ref, bp_ref, wv_ref, bv_ref,
              pol_ref, emb_ref):
    out = jnp.maximum(
        jnp.dot(lat_ref[...], w1_ref[...],
                preferred_element_type=jnp.float32) + b1_ref[...], 0.0)
    pol_ref[...] = jnp.dot(out, wp_ref[...],
                           preferred_element_type=jnp.float32) + bp_ref[...]
    emb_ref[...] = jnp.dot(out, wv_ref[...],
                           preferred_element_type=jnp.float32) + bv_ref[...]


def _dist_body(emb_ref, keys_ref, d2_ref):
    c = pl.program_id(0)
    qb = emb_ref[...]
    kb = keys_ref[...]
    cross = lax.dot_general(qb, kb, (((1,), (1,)), ((), ())),
                            preferred_element_type=jnp.float32)
    q2 = jnp.sum(qb * qb, axis=1, keepdims=True)
    k2 = jnp.sum(kb * kb, axis=1)[None, :]
    d2 = jnp.maximum(q2 + k2 - 2.0 * cross, 0.0)
    gcols = lax.broadcasted_iota(jnp.int32, (B, CB), 1) + c * CB
    d2_ref[...] = jnp.where(gcols < M, d2, jnp.float32(jnp.inf))


def _chunkmin_body(d2_ref, cm_ref):
    cm_ref[...] = jnp.min(d2_ref[...].reshape(RB, NCH, CHW), axis=2)


def _sc_select_body(d2t, cmt, mvt, num_hbm, ksum_hbm,
                    cmbuf, idsd, idsl, chunkbuf, mvbuf,
                    d2lt, mvlt, d2eq, mveq, numbuf, ksumbuf, sem):
    wid = lax.axis_index("s") * NC_SC + lax.axis_index("c")
    iota = lax.iota(jnp.int32, 16)
    one = jnp.int32(1)
    zero = jnp.int32(0)
    zeros_i = jnp.zeros((16,), jnp.int32)
    zeros_f = jnp.zeros((16,), jnp.float32)

    def splat_f(bits_scalar):
        return lax.bitcast_convert_type(
            jnp.full((16,), bits_scalar, jnp.int32), jnp.float32)

    def kern_of(v):
        x = jnp.maximum(v, jnp.float32(1e-12))
        bits = lax.bitcast_convert_type(x, jnp.int32)
        r = lax.bitcast_convert_type(
            jnp.full((16,), jnp.int32(0x5F3759DF), jnp.int32)
            - lax.shift_right_arithmetic(bits, 1), jnp.float32)
        for _ in range(4):
            r = r * (1.5 - 0.5 * x * r * r)
        return 1.0 / (x * r + DELTA)


    _gdn = lax.GatherDimensionNumbers(offset_dims=(),
                                      collapsed_slice_dims=(0,),
                                      start_index_map=(0,))

    def lane_perm(x, idx):
        return lax.gather(x, idx[:, None], _gdn, slice_sizes=(1,),
                          mode=lax.GatherScatterMode.PROMISE_IN_BOUNDS)

    def lane_sum(x):
        for s in (8, 4, 2, 1):
            x = x + lane_perm(x, iota ^ s)
        return x

    fifteen = jnp.full((16,), 15, jnp.int32)

    def prefix_inc(m):
        x = jnp.where(m, one, zero)
        for s in (1, 2, 4, 8):
            shifted = lane_perm(x, jnp.maximum(iota - s, 0))
            x = x + jnp.where(iota >= s, shifted, zero)
        return x

    def row_body(r, carry):
        numacc, ksumacc = carry
        row = wid * RPW + r
        pltpu.sync_copy(cmt.at[pl.ds(row * NCH, NCH)], cmbuf)
        k_spl = jnp.full((16,), K, jnp.int32)

        def cm_count(tb_vec):
            tf = lax.bitcast_convert_type(tb_vec, jnp.float32)
            def cb(i, acc):
                v = cmbuf[pl.ds(i * 16, 16)]
                return acc + jnp.where(v <= tf, one, zero)
            return lane_sum(lax.fori_loop(0, NCH // 16, cb, zeros_i))

        def cm_bs(_, lh):
            lo, hi = lh
            mid = lo + lax.shift_right_arithmetic(hi - lo, 1)
            ge = cm_count(mid) >= k_spl
            return (jnp.where(ge, lo, mid + 1), jnp.where(ge, mid, hi))

        taub = lax.fori_loop(0, 31, cm_bs,
                             (zeros_i, jnp.full((16,), INF_BITS,
                                                jnp.int32)))[1]
        tauf = lax.bitcast_convert_type(taub, jnp.float32)

        def zb(i, _):
            idsd[pl.ds(i * 16, 16)] = zeros_i
            idsl[pl.ds(i * 16, 16)] = zeros_i
            return zero
        lax.fori_loop(0, NCH // 16, zb, zero)
        rowbase = row * NCH

        def sb(i, off):
            v = cmbuf[pl.ds(i * 16, 16)]
            m = v <= tauf
            pre = prefix_inc(m)
            pos = jnp.maximum(off + pre - 1, 0)
            cid = i * 16 + iota
            plsc.store_scatter(idsl, [pos], cid, mask=m)
            plsc.store_scatter(idsd, [pos], cid + rowbase, mask=m)
            return off + lane_perm(pre, fifteen)
        n_sel_v = lax.fori_loop(0, NCH // 16, sb, zeros_i)
        n_sel = n_sel_v[0]

        nb = lax.div(n_sel + 63, jnp.int32(64))

        def gb(b, _):
            base = b * 64
            pltpu.async_copy(d2t.at[idsd.at[pl.ds(base, 64)]],
                             chunkbuf.at[pl.ds(base, 64)], sem).wait()
            pltpu.async_copy(mvt.at[idsl.at[pl.ds(base, 64)]],
                             mvbuf.at[pl.ds(base, 64)], sem).wait()
            return zero
        lax.fori_loop(0, nb, gb, zero)

        def el_count(tb_vec, strict):
            tf = lax.bitcast_convert_type(tb_vec, jnp.float32)
            def cb(s, acc):
                a = acc
                for j in range(8):
                    v = chunkbuf[s, pl.ds(j * 16, 16)]
                    m = (v < tf) if strict else (v <= tf)
                    a = a + jnp.where(m, one, zero)
                return a
            return lane_sum(lax.fori_loop(0, n_sel, cb, zeros_i))

        def el_bs(_, lh):
            lo, hi = lh
            mid = lo + lax.shift_right_arithmetic(hi - lo, 1)
            ge = el_count(mid, False) >= k_spl
            return (jnp.where(ge, lo, mid + 1), jnp.where(ge, mid, hi))

        tb50 = lax.fori_loop(0, 31, el_bs, (zeros_i, taub))[1]
        tf50 = lax.bitcast_convert_type(tb50, jnp.float32)
        cnt_lt = el_count(tb50, True)
        n_eq = k_spl - cnt_lt

        cap_spl = jnp.full((16,), CAP, jnp.int32)

        def col_b(s, offs):
            olt, oeq = offs
            for j in range(8):
                v = chunkbuf[s, pl.ds(j * 16, 16)]
                w = mvbuf[s, pl.ds(j * 16, 16)]
                mlt = v < tf50
                meq = v == tf50
                plt_pre = prefix_inc(mlt)
                peq_pre = prefix_inc(meq)
                plt = olt + plt_pre - 1
                peq = oeq + peq_pre - 1
                wlt = mlt & (plt < cap_spl)
                weq = meq & (peq < cap_spl)
                plt = jnp.clip(plt, 0, CAP - 1)
                peq = jnp.clip(peq, 0, CAP - 1)
                plsc.store_scatter(d2lt, [plt], v, mask=wlt)
                plsc.store_scatter(mvlt, [plt], w, mask=wlt)
                plsc.store_scatter(d2eq, [peq], v, mask=weq)
                plsc.store_scatter(mveq, [peq], w, mask=weq)
                olt = olt + lane_perm(plt_pre, fifteen)
                oeq = oeq + lane_perm(peq_pre, fifteen)
            return (olt, oeq)
        lax.fori_loop(0, n_sel, col_b, (zeros_i, zeros_i))

        accn, acck = zeros_f, zeros_f
        for j in range(CAP // 16):
            pos = j * 16 + iota
            mlt = pos < cnt_lt
            meq = pos < n_eq
            klt = jnp.where(mlt, kern_of(d2lt[pl.ds(j * 16, 16)]), 0.0)
            keq = jnp.where(meq, kern_of(d2eq[pl.ds(j * 16, 16)]), 0.0)
            accn = (accn + klt * jnp.where(mlt, mvlt[pl.ds(j * 16, 16)], 0.0)
                    + keq * jnp.where(meq, mveq[pl.ds(j * 16, 16)], 0.0))
            acck = acck + klt + keq
        num_r = lane_sum(accn)
        ksum_r = lane_sum(acck)
        lane = iota == jnp.full((16,), r, jnp.int32)
        numacc = numacc + jnp.where(lane, num_r, 0.0)
        ksumacc = ksumacc + jnp.where(lane, ksum_r, 0.0)
        return (numacc, ksumacc)

    numacc, ksumacc = lax.fori_loop(0, RPW, row_body, (zeros_f, zeros_f))
    numbuf[...] = numacc
    ksumbuf[...] = ksumacc
    pltpu.sync_copy(numbuf, num_hbm.at[pl.ds(wid * RPW, RPW)])
    pltpu.sync_copy(ksumbuf, ksum_hbm.at[pl.ds(wid * RPW, RPW)])


_sc_select = functools.partial(
    pl.kernel,
    out_type=[jax.ShapeDtypeStruct((B,), jnp.float32),
              jax.ShapeDtypeStruct((B,), jnp.float32)],
    mesh=plsc.VectorSubcoreMesh(core_axis_name="c", subcore_axis_name="s"),
    compiler_params=pltpu.CompilerParams(needs_layout_passes=False),
    scratch_types=[
        pltpu.VMEM((NCH,), jnp.float32),
        pltpu.VMEM((NCH,), jnp.int32),
        pltpu.VMEM((NCH,), jnp.int32),
        pltpu.VMEM((NCH, CHW), jnp.float32),
        pltpu.VMEM((NCH, CHW), jnp.float32),
        pltpu.VMEM((CAP,), jnp.float32),
        pltpu.VMEM((CAP,), jnp.float32),
        pltpu.VMEM((CAP,), jnp.float32),
        pltpu.VMEM((CAP,), jnp.float32),
        pltpu.VMEM((16,), jnp.float32),
        pltpu.VMEM((16,), jnp.float32),
        pltpu.SemaphoreType.DMA,
    ],
)(_sc_select_body)


def _norm_body(num_ref, ksum_ref, val_ref):
    val_ref[...] = num_ref[...] / jnp.sum(ksum_ref[...])


def kernel(latent, W1, b1, Wp, bp, Wv, bv, mem_keys, mem_values):
    policy, emb = pl.pallas_call(
        _mlp_body,
        out_shape=[jax.ShapeDtypeStruct((B, A), jnp.float32),
                   jax.ShapeDtypeStruct((B, D), jnp.float32)],
    )(latent, W1, b1.reshape(1, D), Wp, bp.reshape(1, A), Wv,
      bv.reshape(1, D))

    keys_pad = jnp.pad(mem_keys, ((0, MP - M), (0, 0)))
    mv_pad = jnp.pad(mem_values, (0, MP - M))

    d2 = pl.pallas_call(
        _dist_body,
        grid=(NCB,),
        in_specs=[
            pl.BlockSpec((B, D), lambda c: (0, 0)),
            pl.BlockSpec((CB, D), lambda c: (c, 0)),
        ],
        out_specs=pl.BlockSpec((B, CB), lambda c: (0, c)),
        out_shape=jax.ShapeDtypeStruct((B, MP), jnp.float32),
    )(emb, keys_pad)

    cm = pl.pallas_call(
        _chunkmin_body,
        grid=(NRB,),
        in_specs=[pl.BlockSpec((RB, MP), lambda r: (r, 0))],
        out_specs=pl.BlockSpec((RB, NCH), lambda r: (r, 0)),
        out_shape=jax.ShapeDtypeStruct((B, NCH), jnp.float32),
    )(d2)

    num, ksum = _sc_select(d2.reshape(B * NCH, CHW), cm.reshape(B * NCH),
                           mv_pad.reshape(NCH, CHW))

    value = pl.pallas_call(
        _norm_body,
        out_shape=jax.ShapeDtypeStruct((B, 1), jnp.float32),
    )(num.reshape(B, 1), ksum.reshape(B, 1))

    return policy, value.reshape(B)


# speedup vs baseline: 5.6031x; 1.1835x over previous
import functools

import jax
import jax.numpy as jnp
from jax import lax
from jax.experimental import pallas as pl
from jax.experimental.pallas import tpu as pltpu
from jax.experimental.pallas import tpu_sc as plsc

B, D, A, M, K = 512, 256, 18, 50000, 50
DELTA = 0.001
MP = 51200
CB = 2048
RB = 64
CHW = 128
NCH = MP // CHW
NRB, NCB = B // RB, MP // CB
NC_SC, NS_SC = 2, 16
NW = NC_SC * NS_SC
RPW = B // NW
CAP = 96
CAND_CAP = 256
INF_BITS = 0x7F800000


def _mlp_body(lat_ref, w1_ref, b1_ref, wp_ref, bp_ref, wv_ref, bv_ref,
              pol_ref, emb_ref):
    out = jnp.maximum(
        jnp.dot(lat_ref[...], w1_ref[...],
                preferred_element_type=jnp.float32) + b1_ref[...], 0.0)
    pol_ref[...] = jnp.dot(out, wp_ref[...],
                           preferred_element_type=jnp.float32) + bp_ref[...]
    emb_ref[...] = jnp.dot(out, wv_ref[...],
                           preferred_element_type=jnp.float32) + bv_ref[...]


def _dist_body(emb_ref, keys_ref, d2_ref):
    c = pl.program_id(0)
    qb = emb_ref[...]
    kb = keys_ref[...]
    cross = lax.dot_general(qb, kb, (((1,), (1,)), ((), ())),
                            preferred_element_type=jnp.float32)
    q2 = jnp.sum(qb * qb, axis=1, keepdims=True)
    k2 = jnp.sum(kb * kb, axis=1)[None, :]
    d2 = jnp.maximum(q2 + k2 - 2.0 * cross, 0.0)
    gcols = lax.broadcasted_iota(jnp.int32, (B, CB), 1) + c * CB
    d2_ref[...] = jnp.where(gcols < M, d2, jnp.float32(jnp.inf))


def _chunkmin_body(d2_ref, cm_ref):
    cm_ref[...] = jnp.min(d2_ref[...].reshape(RB, NCH, CHW), axis=2)


def _sc_select_body(d2t, cmt, mvt, num_hbm, ksum_hbm,
                    cmbuf, idsd, idsl, chunkbuf, mvbuf,
                    d2lt, mvlt, d2eq, mveq, candv, candw,
                    numbuf, ksumbuf, sem, sem2):
    wid = lax.axis_index("s") * NC_SC + lax.axis_index("c")
    iota = lax.iota(jnp.int32, 16)
    one = jnp.int32(1)
    zero = jnp.int32(0)
    zeros_i = jnp.zeros((16,), jnp.int32)
    zeros_f = jnp.zeros((16,), jnp.float32)

    def splat_f(bits_scalar):
        return lax.bitcast_convert_type(
            jnp.full((16,), bits_scalar, jnp.int32), jnp.float32)

    def kern_of(v):
        x = jnp.maximum(v, jnp.float32(1e-12))
        bits = lax.bitcast_convert_type(x, jnp.int32)
        r = lax.bitcast_convert_type(
            jnp.full((16,), jnp.int32(0x5F3759DF), jnp.int32)
            - lax.shift_right_arithmetic(bits, 1), jnp.float32)
        for _ in range(4):
            r = r * (1.5 - 0.5 * x * r * r)
        return 1.0 / (x * r + DELTA)


    _gdn = lax.GatherDimensionNumbers(offset_dims=(),
                                      collapsed_slice_dims=(0,),
                                      start_index_map=(0,))

    def lane_perm(x, idx):
        return lax.gather(x, idx[:, None], _gdn, slice_sizes=(1,),
                          mode=lax.GatherScatterMode.PROMISE_IN_BOUNDS)

    def lane_sum(x):
        for s in (8, 4, 2, 1):
            x = x + lane_perm(x, iota ^ s)
        return x

    def lane_min(x):
        for s in (8, 4, 2, 1):
            x = jnp.minimum(x, lane_perm(x, iota ^ s))
        return x

    def lane_max(x):
        for s in (8, 4, 2, 1):
            x = jnp.maximum(x, lane_perm(x, iota ^ s))
        return x

    fifteen = jnp.full((16,), 15, jnp.int32)

    def prefix_inc(m):
        x = jnp.where(m, one, zero)
        for s in (1, 2, 4, 8):
            shifted = lane_perm(x, jnp.maximum(iota - s, 0))
            x = x + jnp.where(iota >= s, shifted, zero)
        return x

    def row_body(r, carry):
        numacc, ksumacc = carry
        row = wid * RPW + r
        pltpu.sync_copy(cmt.at[pl.ds(row * NCH, NCH)], cmbuf)
        k_spl = jnp.full((16,), K, jnp.int32)

        def cm_count(tb_vec):
            tf = lax.bitcast_convert_type(tb_vec, jnp.float32)
            def cb(i, acc):
                v = cmbuf[pl.ds(i * 16, 16)]
                return acc + jnp.where(v <= tf, one, zero)
            return lane_sum(lax.fori_loop(0, NCH // 16, cb, zeros_i))

        def cm_bs(_, lh):
            lo, hi = lh
            mid = lo + lax.shift_right_arithmetic(hi - lo, 1)
            ge = cm_count(mid) >= k_spl
            return (jnp.where(ge, lo, mid + 1), jnp.where(ge, mid, hi))

        def mm_body(i, mm):
            mn, mx = mm
            v = cmbuf[pl.ds(i * 16, 16)]
            return jnp.minimum(mn, v), jnp.maximum(mx, v)
        mn_v, mx_v = lax.fori_loop(0, 24, mm_body,
                                   (jnp.full((16,), jnp.inf, jnp.float32),
                                    zeros_f))
        mn_v = jnp.minimum(mn_v, cmbuf[pl.ds(384, 16)])
        lo_b = lax.bitcast_convert_type(lane_min(mn_v), jnp.int32)
        hi_b = lax.bitcast_convert_type(lane_max(mx_v), jnp.int32)

        taub = lax.fori_loop(0, 31, cm_bs, (lo_b, hi_b))[1]
        tauf = lax.bitcast_convert_type(taub, jnp.float32)

        def zb(i, _):
            idsd[pl.ds(i * 16, 16)] = zeros_i
            idsl[pl.ds(i * 16, 16)] = zeros_i
            return zero
        lax.fori_loop(0, NCH // 16, zb, zero)
        rowbase = row * NCH

        def sb(i, off):
            v = cmbuf[pl.ds(i * 16, 16)]
            m = v <= tauf
            pre = prefix_inc(m)
            pos = jnp.maximum(off + pre - 1, 0)
            cid = i * 16 + iota
            plsc.store_scatter(idsl, [pos], cid, mask=m)
            plsc.store_scatter(idsd, [pos], cid + rowbase, mask=m)
            return off + lane_perm(pre, fifteen)
        n_sel_v = lax.fori_loop(0, NCH // 16, sb, zeros_i)
        n_sel = n_sel_v[0]

        nb = lax.div(n_sel + 63, jnp.int32(64))

        def gb(b, _):
            base = b * 64
            cp1 = pltpu.async_copy(d2t.at[idsd.at[pl.ds(base, 64)]],
                                   chunkbuf.at[pl.ds(base, 64)], sem)
            cp2 = pltpu.async_copy(mvt.at[idsl.at[pl.ds(base, 64)]],
                                   mvbuf.at[pl.ds(base, 64)], sem2)
            cp1.wait()
            cp2.wait()
            return zero
        lax.fori_loop(0, nb, gb, zero)

        ccap_spl = jnp.full((16,), CAND_CAP, jnp.int32)

        def cand_b(s, off):
            o = off
            for j in range(8):
                v = chunkbuf[s, pl.ds(j * 16, 16)]
                w = mvbuf[s, pl.ds(j * 16, 16)]
                m = v <= tauf
                pre = prefix_inc(m)
                pos = o + pre - 1
                wm = m & (pos < ccap_spl)
                pos = jnp.clip(pos, 0, CAND_CAP - 1)
                plsc.store_scatter(candv, [pos], v, mask=wm)
                plsc.store_scatter(candw, [pos], w, mask=wm)
                o = o + lane_perm(pre, fifteen)
            return o
        n_cand_v = lax.fori_loop(0, n_sel, cand_b, zeros_i)

        def rank_and_combine(load_v, load_w, nvalid_v, n_vregs):
            def cnt(tb_vec, strict):
                tf = lax.bitcast_convert_type(tb_vec, jnp.float32)
                def cb(i, acc):
                    v = load_v(i)
                    m = (v < tf) if strict else (v <= tf)
                    m = m & ((i * 16 + iota) < nvalid_v)
                    return acc + jnp.where(m, one, zero)
                return lane_sum(lax.fori_loop(0, n_vregs, cb, zeros_i))

            def bs(_, lh):
                lo, hi = lh
                mid = lo + lax.shift_right_arithmetic(hi - lo, 1)
                ge = cnt(mid, False) >= k_spl
                return (jnp.where(ge, lo, mid + 1), jnp.where(ge, mid, hi))

            tb50 = lax.fori_loop(0, 31, bs, (zeros_i, taub))[1]
            tf50 = lax.bitcast_convert_type(tb50, jnp.float32)
            n_eq = k_spl - cnt(tb50, True)

            def fin_b(i, c):
                accn, acck, eqc = c
                v = load_v(i)
                w = load_w(i)
                valid = (i * 16 + iota) < nvalid_v
                meq = (v == tf50) & valid
                pre = prefix_inc(meq) + eqc
                m = ((v < tf50) & valid) | (meq & (pre <= n_eq))
                kv = jnp.where(m, kern_of(v), 0.0)
                accn = accn + kv * jnp.where(m, w, 0.0)
                acck = acck + kv
                return (accn, acck, lane_perm(pre, fifteen))
            accn, acck, _ = lax.fori_loop(0, n_vregs, fin_b,
                                          (zeros_f, zeros_f, zeros_i))
            return lane_sum(accn), lane_sum(acck)

        def fast_path(_):
            nv = lax.div(n_cand_v[0] + 15, jnp.int32(16))
            return rank_and_combine(
                lambda i: candv[pl.ds(i * 16, 16)],
                lambda i: candw[pl.ds(i * 16, 16)],
                n_cand_v, nv)

        def slow_path(_):
            def lv(i):
                return chunkbuf[lax.shift_right_arithmetic(i, 3),
                                pl.ds((i & 7) * 16, 16)]
            def lw(i):
                return mvbuf[lax.shift_right_arithmetic(i, 3),
                             pl.ds((i & 7) * 16, 16)]
            return rank_and_combine(lv, lw, n_sel_v * CHW, n_sel * 8)

        num_r, ksum_r = lax.cond(n_cand_v[0] <= CAND_CAP,
                                 fast_path, slow_path, zero)
        lane = iota == jnp.full((16,), r, jnp.int32)
        numacc = numacc + jnp.where(lane, num_r, 0.0)
        ksumacc = ksumacc + jnp.where(lane, ksum_r, 0.0)
        return (numacc, ksumacc)

    numacc, ksumacc = lax.fori_loop(0, RPW, row_body, (zeros_f, zeros_f))
    numbuf[...] = numacc
    ksumbuf[...] = ksumacc
    pltpu.sync_copy(numbuf, num_hbm.at[pl.ds(wid * RPW, RPW)])
    pltpu.sync_copy(ksumbuf, ksum_hbm.at[pl.ds(wid * RPW, RPW)])


_sc_select = functools.partial(
    pl.kernel,
    out_type=[jax.ShapeDtypeStruct((B,), jnp.float32),
              jax.ShapeDtypeStruct((B,), jnp.float32)],
    mesh=plsc.VectorSubcoreMesh(core_axis_name="c", subcore_axis_name="s"),
    compiler_params=pltpu.CompilerParams(needs_layout_passes=False),
    scratch_types=[
        pltpu.VMEM((NCH,), jnp.float32),
        pltpu.VMEM((NCH,), jnp.int32),
        pltpu.VMEM((NCH,), jnp.int32),
        pltpu.VMEM((NCH, CHW), jnp.float32),
        pltpu.VMEM((NCH, CHW), jnp.float32),
        pltpu.VMEM((CAP,), jnp.float32),
        pltpu.VMEM((CAP,), jnp.float32),
        pltpu.VMEM((CAP,), jnp.float32),
        pltpu.VMEM((CAP,), jnp.float32),
        pltpu.VMEM((CAND_CAP,), jnp.float32),
        pltpu.VMEM((CAND_CAP,), jnp.float32),
        pltpu.VMEM((16,), jnp.float32),
        pltpu.VMEM((16,), jnp.float32),
        pltpu.SemaphoreType.DMA,
        pltpu.SemaphoreType.DMA,
    ],
)(_sc_select_body)


def _norm_body(num_ref, ksum_ref, val_ref):
    val_ref[...] = num_ref[...] / jnp.sum(ksum_ref[...])


def kernel(latent, W1, b1, Wp, bp, Wv, bv, mem_keys, mem_values):
    policy, emb = pl.pallas_call(
        _mlp_body,
        out_shape=[jax.ShapeDtypeStruct((B, A), jnp.float32),
                   jax.ShapeDtypeStruct((B, D), jnp.float32)],
    )(latent, W1, b1.reshape(1, D), Wp, bp.reshape(1, A), Wv,
      bv.reshape(1, D))

    keys_pad = jnp.pad(mem_keys, ((0, MP - M), (0, 0)))
    mv_pad = jnp.pad(mem_values, (0, MP - M))

    d2 = pl.pallas_call(
        _dist_body,
        grid=(NCB,),
        in_specs=[
            pl.BlockSpec((B, D), lambda c: (0, 0)),
            pl.BlockSpec((CB, D), lambda c: (c, 0)),
        ],
        out_specs=pl.BlockSpec((B, CB), lambda c: (0, c)),
        out_shape=jax.ShapeDtypeStruct((B, MP), jnp.float32),
    )(emb, keys_pad)

    cm = pl.pallas_call(
        _chunkmin_body,
        grid=(NRB,),
        in_specs=[pl.BlockSpec((RB, MP), lambda r: (r, 0))],
        out_specs=pl.BlockSpec((RB, NCH), lambda r: (r, 0)),
        out_shape=jax.ShapeDtypeStruct((B, NCH), jnp.float32),
    )(d2)

    num, ksum = _sc_select(d2.reshape(B * NCH, CHW), cm.reshape(B * NCH),
                           mv_pad.reshape(NCH, CHW))

    value = pl.pallas_call(
        _norm_body,
        out_shape=jax.ShapeDtypeStruct((B, 1), jnp.float32),
    )(num.reshape(B, 1), ksum.reshape(B, 1))

    return policy, value.reshape(B)


# speedup vs baseline: 5.6421x; 1.0070x over previous
import functools

import jax
import jax.numpy as jnp
from jax import lax
from jax.experimental import pallas as pl
from jax.experimental.pallas import tpu as pltpu
from jax.experimental.pallas import tpu_sc as plsc

B, D, A, M, K = 512, 256, 18, 50000, 50
DELTA = 0.001
MP = 51200
CB = 2048
RB = 64
CHW = 128
NCH = MP // CHW
NRB, NCB = B // RB, MP // CB
NC_SC, NS_SC = 2, 16
NW = NC_SC * NS_SC
RPW = B // NW
CAP = 96
CAND_CAP = 256
INF_BITS = 0x7F800000


def _mlp_body(lat_ref, w1_ref, b1_ref, wp_ref, bp_ref, wv_ref, bv_ref,
              pol_ref, emb_ref):
    out = jnp.maximum(
        jnp.dot(lat_ref[...], w1_ref[...],
                preferred_element_type=jnp.float32) + b1_ref[...], 0.0)
    pol_ref[...] = jnp.dot(out, wp_ref[...],
                           preferred_element_type=jnp.float32) + bp_ref[...]
    emb_ref[...] = jnp.dot(out, wv_ref[...],
                           preferred_element_type=jnp.float32) + bv_ref[...]


def _dist_body(emb_ref, keys_ref, d2_ref):
    c = pl.program_id(0)
    qb = emb_ref[...]
    kb = keys_ref[...]
    cross = lax.dot_general(qb, kb, (((1,), (1,)), ((), ())),
                            preferred_element_type=jnp.float32)
    q2 = jnp.sum(qb * qb, axis=1, keepdims=True)
    k2 = jnp.sum(kb * kb, axis=1)[None, :]
    d2 = jnp.maximum(q2 + k2 - 2.0 * cross, 0.0)
    gcols = lax.broadcasted_iota(jnp.int32, (B, CB), 1) + c * CB
    d2_ref[...] = jnp.where(gcols < M, d2, jnp.float32(jnp.inf))


def _chunkmin_body(d2_ref, cm_ref):
    cm_ref[...] = jnp.min(d2_ref[...].reshape(RB, NCH, CHW), axis=2)


def _sc_select_body(d2t, cmt, mvt, num_hbm, ksum_hbm,
                    cmbuf, idsd, idsl, chunkbuf, mvbuf,
                    d2lt, mvlt, d2eq, mveq, candv, candw,
                    numbuf, ksumbuf, sem, sem2):
    wid = lax.axis_index("s") * NC_SC + lax.axis_index("c")
    iota = lax.iota(jnp.int32, 16)
    one = jnp.int32(1)
    zero = jnp.int32(0)
    zeros_i = jnp.zeros((16,), jnp.int32)
    zeros_f = jnp.zeros((16,), jnp.float32)

    def splat_f(bits_scalar):
        return lax.bitcast_convert_type(
            jnp.full((16,), bits_scalar, jnp.int32), jnp.float32)

    def kern_of(v):
        x = jnp.maximum(v, jnp.float32(1e-12))
        bits = lax.bitcast_convert_type(x, jnp.int32)
        r = lax.bitcast_convert_type(
            jnp.full((16,), jnp.int32(0x5F3759DF), jnp.int32)
            - lax.shift_right_arithmetic(bits, 1), jnp.float32)
        for _ in range(4):
            r = r * (1.5 - 0.5 * x * r * r)
        return 1.0 / (x * r + DELTA)


    _gdn = lax.GatherDimensionNumbers(offset_dims=(),
                                      collapsed_slice_dims=(0,),
                                      start_index_map=(0,))

    def lane_perm(x, idx):
        return lax.gather(x, idx[:, None], _gdn, slice_sizes=(1,),
                          mode=lax.GatherScatterMode.PROMISE_IN_BOUNDS)

    def lane_sum(x):
        for s in (8, 4, 2, 1):
            x = x + lane_perm(x, iota ^ s)
        return x

    def lane_min(x):
        for s in (8, 4, 2, 1):
            x = jnp.minimum(x, lane_perm(x, iota ^ s))
        return x

    def lane_max(x):
        for s in (8, 4, 2, 1):
            x = jnp.maximum(x, lane_perm(x, iota ^ s))
        return x

    def prefix_inc(m):
        return plsc.cumsum(jnp.where(m, one, zero))

    def popc(m):
        return plsc.all_reduce_population_count(m)

    def row_body(r, carry):
        numacc, ksumacc = carry
        row = wid * RPW + r
        pltpu.sync_copy(cmt.at[pl.ds(row * NCH, NCH)], cmbuf)
        k_spl = jnp.full((16,), K, jnp.int32)

        def cm_count(tb_vec):
            tf = lax.bitcast_convert_type(tb_vec, jnp.float32)
            def cb(i, acc):
                v = cmbuf[pl.ds(i * 16, 16)]
                return acc + jnp.where(v <= tf, one, zero)
            return lane_sum(lax.fori_loop(0, NCH // 16, cb, zeros_i))

        def cm_bs(_, lh):
            lo, hi = lh
            mid = lo + lax.shift_right_arithmetic(hi - lo, 1)
            ge = cm_count(mid) >= k_spl
            return (jnp.where(ge, lo, mid + 1), jnp.where(ge, mid, hi))

        def mm_body(i, mm):
            mn, mx = mm
            v = cmbuf[pl.ds(i * 16, 16)]
            return jnp.minimum(mn, v), jnp.maximum(mx, v)
        mn_v, mx_v = lax.fori_loop(0, 24, mm_body,
                                   (jnp.full((16,), jnp.inf, jnp.float32),
                                    zeros_f))
        mn_v = jnp.minimum(mn_v, cmbuf[pl.ds(384, 16)])
        lo_b = lax.bitcast_convert_type(lane_min(mn_v), jnp.int32)
        hi_b = lax.bitcast_convert_type(lane_max(mx_v), jnp.int32)

        taub = lax.fori_loop(0, 31, cm_bs, (lo_b, hi_b))[1]
        tauf = lax.bitcast_convert_type(taub, jnp.float32)

        def zb(i, _):
            idsd[pl.ds(i * 16, 16)] = zeros_i
            idsl[pl.ds(i * 16, 16)] = zeros_i
            return zero
        lax.fori_loop(0, NCH // 16, zb, zero)
        rowbase = row * NCH

        def sb(i, off):
            v = cmbuf[pl.ds(i * 16, 16)]
            m = v <= tauf
            pre = prefix_inc(m)
            pos = jnp.maximum(off + pre - 1, 0)
            cid = i * 16 + iota
            plsc.store_scatter(idsl, [pos], cid, mask=m)
            plsc.store_scatter(idsd, [pos], cid + rowbase, mask=m)
            return off + popc(m)
        n_sel_v = lax.fori_loop(0, NCH // 16, sb, zeros_i)
        n_sel = n_sel_v[0]

        nb = lax.div(n_sel + 63, jnp.int32(64))

        def gb(b, _):
            base = b * 64
            cp1 = pltpu.async_copy(d2t.at[idsd.at[pl.ds(base, 64)]],
                                   chunkbuf.at[pl.ds(base, 64)], sem)
            cp2 = pltpu.async_copy(mvt.at[idsl.at[pl.ds(base, 64)]],
                                   mvbuf.at[pl.ds(base, 64)], sem2)
            cp1.wait()
            cp2.wait()
            return zero
        lax.fori_loop(0, nb, gb, zero)

        ccap_spl = jnp.full((16,), CAND_CAP, jnp.int32)

        def cand_b(s, off):
            o = off
            for j in range(8):
                v = chunkbuf[s, pl.ds(j * 16, 16)]
                w = mvbuf[s, pl.ds(j * 16, 16)]
                m = v <= tauf
                pre = prefix_inc(m)
                pos = o + pre - 1
                wm = m & (pos < ccap_spl)
                pos = jnp.clip(pos, 0, CAND_CAP - 1)
                plsc.store_scatter(candv, [pos], v, mask=wm)
                plsc.store_scatter(candw, [pos], w, mask=wm)
                o = o + popc(m)
            return o
        n_cand_v = lax.fori_loop(0, n_sel, cand_b, zeros_i)

        def rank_and_combine(load_v, load_w, nvalid_v, n_vregs):
            def cnt(tb_vec, strict):
                tf = lax.bitcast_convert_type(tb_vec, jnp.float32)
                def cb(i, acc):
                    v = load_v(i)
                    m = (v < tf) if strict else (v <= tf)
                    m = m & ((i * 16 + iota) < nvalid_v)
                    return acc + jnp.where(m, one, zero)
                return lane_sum(lax.fori_loop(0, n_vregs, cb, zeros_i))

            def bs(_, lh):
                lo, hi = lh
                mid = lo + lax.shift_right_arithmetic(hi - lo, 1)
                ge = cnt(mid, False) >= k_spl
                return (jnp.where(ge, lo, mid + 1), jnp.where(ge, mid, hi))

            tb50 = lax.fori_loop(0, 31, bs, (zeros_i, taub))[1]
            tf50 = lax.bitcast_convert_type(tb50, jnp.float32)
            n_eq = k_spl - cnt(tb50, True)

            def fin_b(i, c):
                accn, acck, eqc = c
                v = load_v(i)
                w = load_w(i)
                valid = (i * 16 + iota) < nvalid_v
                meq = (v == tf50) & valid
                pre = prefix_inc(meq) + eqc
                m = ((v < tf50) & valid) | (meq & (pre <= n_eq))
                kv = jnp.where(m, kern_of(v), 0.0)
                accn = accn + kv * jnp.where(m, w, 0.0)
                acck = acck + kv
                return (accn, acck, eqc + popc(meq))
            accn, acck, _ = lax.fori_loop(0, n_vregs, fin_b,
                                          (zeros_f, zeros_f, zeros_i))
            return lane_sum(accn), lane_sum(acck)

        def fast_path(_):
            nv = lax.div(n_cand_v[0] + 15, jnp.int32(16))
            return rank_and_combine(
                lambda i: candv[pl.ds(i * 16, 16)],
                lambda i: candw[pl.ds(i * 16, 16)],
                n_cand_v, nv)

        def slow_path(_):
            def lv(i):
                return chunkbuf[lax.shift_right_arithmetic(i, 3),
                                pl.ds((i & 7) * 16, 16)]
            def lw(i):
                return mvbuf[lax.shift_right_arithmetic(i, 3),
                             pl.ds((i & 7) * 16, 16)]
            return rank_and_combine(lv, lw, n_sel_v * CHW, n_sel * 8)

        num_r, ksum_r = lax.cond(n_cand_v[0] <= CAND_CAP,
                                 fast_path, slow_path, zero)
        lane = iota == jnp.full((16,), r, jnp.int32)
        numacc = numacc + jnp.where(lane, num_r, 0.0)
        ksumacc = ksumacc + jnp.where(lane, ksum_r, 0.0)
        return (numacc, ksumacc)

    numacc, ksumacc = lax.fori_loop(0, RPW, row_body, (zeros_f, zeros_f))
    numbuf[...] = numacc
    ksumbuf[...] = ksumacc
    pltpu.sync_copy(numbuf, num_hbm.at[pl.ds(wid * RPW, RPW)])
    pltpu.sync_copy(ksumbuf, ksum_hbm.at[pl.ds(wid * RPW, RPW)])


_sc_select = functools.partial(
    pl.kernel,
    out_type=[jax.ShapeDtypeStruct((B,), jnp.float32),
              jax.ShapeDtypeStruct((B,), jnp.float32)],
    mesh=plsc.VectorSubcoreMesh(core_axis_name="c", subcore_axis_name="s"),
    compiler_params=pltpu.CompilerParams(needs_layout_passes=False),
    scratch_types=[
        pltpu.VMEM((NCH,), jnp.float32),
        pltpu.VMEM((NCH,), jnp.int32),
        pltpu.VMEM((NCH,), jnp.int32),
        pltpu.VMEM((NCH, CHW), jnp.float32),
        pltpu.VMEM((NCH, CHW), jnp.float32),
        pltpu.VMEM((CAP,), jnp.float32),
        pltpu.VMEM((CAP,), jnp.float32),
        pltpu.VMEM((CAP,), jnp.float32),
        pltpu.VMEM((CAP,), jnp.float32),
        pltpu.VMEM((CAND_CAP,), jnp.float32),
        pltpu.VMEM((CAND_CAP,), jnp.float32),
        pltpu.VMEM((16,), jnp.float32),
        pltpu.VMEM((16,), jnp.float32),
        pltpu.SemaphoreType.DMA,
        pltpu.SemaphoreType.DMA,
    ],
)(_sc_select_body)


def _norm_body(num_ref, ksum_ref, val_ref):
    val_ref[...] = num_ref[...] / jnp.sum(ksum_ref[...])


def kernel(latent, W1, b1, Wp, bp, Wv, bv, mem_keys, mem_values):
    policy, emb = pl.pallas_call(
        _mlp_body,
        out_shape=[jax.ShapeDtypeStruct((B, A), jnp.float32),
                   jax.ShapeDtypeStruct((B, D), jnp.float32)],
    )(latent, W1, b1.reshape(1, D), Wp, bp.reshape(1, A), Wv,
      bv.reshape(1, D))

    keys_pad = jnp.pad(mem_keys, ((0, MP - M), (0, 0)))
    mv_pad = jnp.pad(mem_values, (0, MP - M))

    d2 = pl.pallas_call(
        _dist_body,
        grid=(NCB,),
        in_specs=[
            pl.BlockSpec((B, D), lambda c: (0, 0)),
            pl.BlockSpec((CB, D), lambda c: (c, 0)),
        ],
        out_specs=pl.BlockSpec((B, CB), lambda c: (0, c)),
        out_shape=jax.ShapeDtypeStruct((B, MP), jnp.float32),
    )(emb, keys_pad)

    cm = pl.pallas_call(
        _chunkmin_body,
        grid=(NRB,),
        in_specs=[pl.BlockSpec((RB, MP), lambda r: (r, 0))],
        out_specs=pl.BlockSpec((RB, NCH), lambda r: (r, 0)),
        out_shape=jax.ShapeDtypeStruct((B, NCH), jnp.float32),
    )(d2)

    num, ksum = _sc_select(d2.reshape(B * NCH, CHW), cm.reshape(B * NCH),
                           mv_pad.reshape(NCH, CHW))

    value = pl.pallas_call(
        _norm_body,
        out_shape=jax.ShapeDtypeStruct((B, 1), jnp.float32),
    )(num.reshape(B, 1), ksum.reshape(B, 1))

    return policy, value.reshape(B)


# speedup vs baseline: 5.8772x; 1.0417x over previous
import functools

import jax
import jax.numpy as jnp
from jax import lax
from jax.experimental import pallas as pl
from jax.experimental.pallas import tpu as pltpu
from jax.experimental.pallas import tpu_sc as plsc

B, D, A, M, K = 512, 256, 18, 50000, 50
DELTA = 0.001
MP = 51200
CB = 2048
RB = 64
CHW = 128
NCH = MP // CHW
NRB, NCB = B // RB, MP // CB
NC_SC, NS_SC = 2, 16
NW = NC_SC * NS_SC
RPW = B // NW
CAP = 96
CAND_CAP = 256
INF_BITS = 0x7F800000


def _mlp_body(lat_ref, w1_ref, b1_ref, wp_ref, bp_ref, wv_ref, bv_ref,
              pol_ref, emb_ref):
    out = jnp.maximum(
        jnp.dot(lat_ref[...], w1_ref[...],
                preferred_element_type=jnp.float32) + b1_ref[...], 0.0)
    pol_ref[...] = jnp.dot(out, wp_ref[...],
                           preferred_element_type=jnp.float32) + bp_ref[...]
    emb_ref[...] = jnp.dot(out, wv_ref[...],
                           preferred_element_type=jnp.float32) + bv_ref[...]


def _dist_body(emb_ref, keys_ref, d2_ref):
    c = pl.program_id(0)
    qb = emb_ref[...]
    kb = keys_ref[...]
    cross = lax.dot_general(qb, kb, (((1,), (1,)), ((), ())),
                            preferred_element_type=jnp.float32)
    q2 = jnp.sum(qb * qb, axis=1, keepdims=True)
    k2 = jnp.sum(kb * kb, axis=1)[None, :]
    d2 = jnp.maximum(q2 + k2 - 2.0 * cross, 0.0)
    gcols = lax.broadcasted_iota(jnp.int32, (B, CB), 1) + c * CB
    d2_ref[...] = jnp.where(gcols < M, d2, jnp.float32(jnp.inf))


def _chunkmin_body(d2_ref, cm_ref):
    cm_ref[...] = jnp.min(d2_ref[...].reshape(RB, NCH, CHW), axis=2)


def _sc_select_body(d2t, cmt, mvt, num_hbm, ksum_hbm,
                    cmbuf, idsd, idsl, chunkbuf, mvbuf,
                    d2lt, mvlt, d2eq, mveq, candv, candw,
                    numbuf, ksumbuf, sem, sem2):
    wid = lax.axis_index("s") * NC_SC + lax.axis_index("c")
    iota = lax.iota(jnp.int32, 16)
    one = jnp.int32(1)
    zero = jnp.int32(0)
    zeros_i = jnp.zeros((16,), jnp.int32)
    zeros_f = jnp.zeros((16,), jnp.float32)

    def splat_f(bits_scalar):
        return lax.bitcast_convert_type(
            jnp.full((16,), bits_scalar, jnp.int32), jnp.float32)

    def kern_of(v):
        x = jnp.maximum(v, jnp.float32(1e-12))
        bits = lax.bitcast_convert_type(x, jnp.int32)
        r = lax.bitcast_convert_type(
            jnp.full((16,), jnp.int32(0x5F3759DF), jnp.int32)
            - lax.shift_right_arithmetic(bits, 1), jnp.float32)
        for _ in range(4):
            r = r * (1.5 - 0.5 * x * r * r)
        return 1.0 / (x * r + DELTA)


    _gdn = lax.GatherDimensionNumbers(offset_dims=(),
                                      collapsed_slice_dims=(0,),
                                      start_index_map=(0,))

    def lane_perm(x, idx):
        return lax.gather(x, idx[:, None], _gdn, slice_sizes=(1,),
                          mode=lax.GatherScatterMode.PROMISE_IN_BOUNDS)

    def lane_sum(x):
        for s in (8, 4, 2, 1):
            x = x + lane_perm(x, iota ^ s)
        return x

    def lane_min(x):
        for s in (8, 4, 2, 1):
            x = jnp.minimum(x, lane_perm(x, iota ^ s))
        return x

    def lane_max(x):
        for s in (8, 4, 2, 1):
            x = jnp.maximum(x, lane_perm(x, iota ^ s))
        return x

    def prefix_inc(m):
        return plsc.cumsum(jnp.where(m, one, zero))

    def popc(m):
        return plsc.all_reduce_population_count(m)

    def row_body(r, carry):
        numacc, ksumacc = carry
        row = wid * RPW + r
        pltpu.sync_copy(cmt.at[pl.ds(row * NCH, NCH)], cmbuf)
        k_spl = jnp.full((16,), K, jnp.int32)

        def cm_count(tb_vec):
            tf = lax.bitcast_convert_type(tb_vec, jnp.float32)
            def cb(i, acc):
                v = cmbuf[pl.ds(i * 16, 16)]
                return acc + jnp.where(v <= tf, one, zero)
            return lane_sum(lax.fori_loop(0, NCH // 16, cb, zeros_i))

        def cm_bs(_, lh):
            lo, hi = lh
            mid = lo + lax.shift_right_arithmetic(hi - lo, 1)
            ge = cm_count(mid) >= k_spl
            return (jnp.where(ge, lo, mid + 1), jnp.where(ge, mid, hi))

        def mm_body(i, mm):
            mn, mx = mm
            v = cmbuf[pl.ds(i * 16, 16)]
            return jnp.minimum(mn, v), jnp.maximum(mx, v)
        mn_v, mx_v = lax.fori_loop(0, 24, mm_body,
                                   (jnp.full((16,), jnp.inf, jnp.float32),
                                    zeros_f))
        mn_v = jnp.minimum(mn_v, cmbuf[pl.ds(384, 16)])
        lo_b = lax.bitcast_convert_type(lane_min(mn_v), jnp.int32)
        hi_b = lax.bitcast_convert_type(lane_max(mx_v), jnp.int32)

        taub = lax.fori_loop(0, 8, cm_bs, (lo_b, hi_b))[1]
        tauf = lax.bitcast_convert_type(taub, jnp.float32)

        def zb(i, _):
            idsd[pl.ds(i * 16, 16)] = zeros_i
            idsl[pl.ds(i * 16, 16)] = zeros_i
            return zero
        lax.fori_loop(0, NCH // 16, zb, zero)
        rowbase = row * NCH

        def sb(i, off):
            v = cmbuf[pl.ds(i * 16, 16)]
            m = v <= tauf
            pre = prefix_inc(m)
            pos = jnp.maximum(off + pre - 1, 0)
            cid = i * 16 + iota
            plsc.store_scatter(idsl, [pos], cid, mask=m)
            plsc.store_scatter(idsd, [pos], cid + rowbase, mask=m)
            return off + popc(m)
        n_sel_v = lax.fori_loop(0, NCH // 16, sb, zeros_i)
        n_sel = n_sel_v[0]

        nb = lax.div(n_sel + 63, jnp.int32(64))

        def gb(b, _):
            base = b * 64
            cp1 = pltpu.async_copy(d2t.at[idsd.at[pl.ds(base, 64)]],
                                   chunkbuf.at[pl.ds(base, 64)], sem)
            cp2 = pltpu.async_copy(mvt.at[idsl.at[pl.ds(base, 64)]],
                                   mvbuf.at[pl.ds(base, 64)], sem2)
            cp1.wait()
            cp2.wait()
            return zero
        lax.fori_loop(0, nb, gb, zero)

        ccap_spl = jnp.full((16,), CAND_CAP, jnp.int32)

        def cand_b(s, off):
            o = off
            for j in range(8):
                v = chunkbuf[s, pl.ds(j * 16, 16)]
                w = mvbuf[s, pl.ds(j * 16, 16)]
                m = v <= tauf
                pre = prefix_inc(m)
                pos = o + pre - 1
                wm = m & (pos < ccap_spl)
                pos = jnp.clip(pos, 0, CAND_CAP - 1)
                plsc.store_scatter(candv, [pos], v, mask=wm)
                plsc.store_scatter(candw, [pos], w, mask=wm)
                o = o + popc(m)
            return o
        n_cand_v = lax.fori_loop(0, n_sel, cand_b, zeros_i)

        def rank_and_combine(load_v, load_w, nvalid_v, n_vregs):
            def cnt(tb_vec, strict):
                tf = lax.bitcast_convert_type(tb_vec, jnp.float32)
                def cb(i, acc):
                    v = load_v(i)
                    m = (v < tf) if strict else (v <= tf)
                    m = m & ((i * 16 + iota) < nvalid_v)
                    return acc + jnp.where(m, one, zero)
                return lane_sum(lax.fori_loop(0, n_vregs, cb, zeros_i))

            def bs(_, lh):
                lo, hi = lh
                mid = lo + lax.shift_right_arithmetic(hi - lo, 1)
                ge = cnt(mid, False) >= k_spl
                return (jnp.where(ge, lo, mid + 1), jnp.where(ge, mid, hi))

            tb50 = lax.fori_loop(0, 31, bs, (lo_b, taub))[1]
            tf50 = lax.bitcast_convert_type(tb50, jnp.float32)
            n_eq = k_spl - cnt(tb50, True)

            def fin_b(i, c):
                accn, acck, eqc = c
                v = load_v(i)
                w = load_w(i)
                valid = (i * 16 + iota) < nvalid_v
                meq = (v == tf50) & valid
                pre = prefix_inc(meq) + eqc
                m = ((v < tf50) & valid) | (meq & (pre <= n_eq))
                kv = jnp.where(m, kern_of(v), 0.0)
                accn = accn + kv * jnp.where(m, w, 0.0)
                acck = acck + kv
                return (accn, acck, eqc + popc(meq))
            accn, acck, _ = lax.fori_loop(0, n_vregs, fin_b,
                                          (zeros_f, zeros_f, zeros_i))
            return lane_sum(accn), lane_sum(acck)

        def fast_path(_):
            nv = lax.div(n_cand_v[0] + 15, jnp.int32(16))
            return rank_and_combine(
                lambda i: candv[pl.ds(i * 16, 16)],
                lambda i: candw[pl.ds(i * 16, 16)],
                n_cand_v, nv)

        def slow_path(_):
            def lv(i):
                return chunkbuf[lax.shift_right_arithmetic(i, 3),
                                pl.ds((i & 7) * 16, 16)]
            def lw(i):
                return mvbuf[lax.shift_right_arithmetic(i, 3),
                             pl.ds((i & 7) * 16, 16)]
            return rank_and_combine(lv, lw, n_sel_v * CHW, n_sel * 8)

        num_r, ksum_r = lax.cond(n_cand_v[0] <= CAND_CAP,
                                 fast_path, slow_path, zero)
        lane = iota == jnp.full((16,), r, jnp.int32)
        numacc = numacc + jnp.where(lane, num_r, 0.0)
        ksumacc = ksumacc + jnp.where(lane, ksum_r, 0.0)
        return (numacc, ksumacc)

    numacc, ksumacc = lax.fori_loop(0, RPW, row_body, (zeros_f, zeros_f))
    numbuf[...] = numacc
    ksumbuf[...] = ksumacc
    pltpu.sync_copy(numbuf, num_hbm.at[pl.ds(wid * RPW, RPW)])
    pltpu.sync_copy(ksumbuf, ksum_hbm.at[pl.ds(wid * RPW, RPW)])


_sc_select = functools.partial(
    pl.kernel,
    out_type=[jax.ShapeDtypeStruct((B,), jnp.float32),
              jax.ShapeDtypeStruct((B,), jnp.float32)],
    mesh=plsc.VectorSubcoreMesh(core_axis_name="c", subcore_axis_name="s"),
    compiler_params=pltpu.CompilerParams(needs_layout_passes=False),
    scratch_types=[
        pltpu.VMEM((NCH,), jnp.float32),
        pltpu.VMEM((NCH,), jnp.int32),
        pltpu.VMEM((NCH,), jnp.int32),
        pltpu.VMEM((NCH, CHW), jnp.float32),
        pltpu.VMEM((NCH, CHW), jnp.float32),
        pltpu.VMEM((CAP,), jnp.float32),
        pltpu.VMEM((CAP,), jnp.float32),
        pltpu.VMEM((CAP,), jnp.float32),
        pltpu.VMEM((CAP,), jnp.float32),
        pltpu.VMEM((CAND_CAP,), jnp.float32),
        pltpu.VMEM((CAND_CAP,), jnp.float32),
        pltpu.VMEM((16,), jnp.float32),
        pltpu.VMEM((16,), jnp.float32),
        pltpu.SemaphoreType.DMA,
        pltpu.SemaphoreType.DMA,
    ],
)(_sc_select_body)


def _norm_body(num_ref, ksum_ref, val_ref):
    val_ref[...] = num_ref[...] / jnp.sum(ksum_ref[...])


def kernel(latent, W1, b1, Wp, bp, Wv, bv, mem_keys, mem_values):
    policy, emb = pl.pallas_call(
        _mlp_body,
        out_shape=[jax.ShapeDtypeStruct((B, A), jnp.float32),
                   jax.ShapeDtypeStruct((B, D), jnp.float32)],
    )(latent, W1, b1.reshape(1, D), Wp, bp.reshape(1, A), Wv,
      bv.reshape(1, D))

    keys_pad = jnp.pad(mem_keys, ((0, MP - M), (0, 0)))
    mv_pad = jnp.pad(mem_values, (0, MP - M))

    d2 = pl.pallas_call(
        _dist_body,
        grid=(NCB,),
        in_specs=[
            pl.BlockSpec((B, D), lambda c: (0, 0)),
            pl.BlockSpec((CB, D), lambda c: (c, 0)),
        ],
        out_specs=pl.BlockSpec((B, CB), lambda c: (0, c)),
        out_shape=jax.ShapeDtypeStruct((B, MP), jnp.float32),
    )(emb, keys_pad)

    cm = pl.pallas_call(
        _chunkmin_body,
        grid=(NRB,),
        in_specs=[pl.BlockSpec((RB, MP), lambda r: (r, 0))],
        out_specs=pl.BlockSpec((RB, NCH), lambda r: (r, 0)),
        out_shape=jax.ShapeDtypeStruct((B, NCH), jnp.float32),
    )(d2)

    num, ksum = _sc_select(d2.reshape(B * NCH, CHW), cm.reshape(B * NCH),
                           mv_pad.reshape(NCH, CHW))

    value = pl.pallas_call(
        _norm_body,
        out_shape=jax.ShapeDtypeStruct((B, 1), jnp.float32),
    )(num.reshape(B, 1), ksum.reshape(B, 1))

    return policy, value.reshape(B)
